# Initial kernel scaffold; baseline (speedup 1.0000x reference)
import jax
import jax.numpy as jnp
from jax.experimental import pallas as pl


def kernel(x, edge_index, W1, b1, W2, b2):
    raise NotImplementedError("write your pallas kernel here")


# speedup vs baseline: 32.5813x; 32.5813x over previous
import functools

import jax
import jax.numpy as jnp
from jax import lax
from jax.experimental import pallas as pl
from jax.experimental.pallas import tpu as pltpu
from jax.experimental.pallas import tpu_sc as plsc

N = 10000
E = 320000
IN_C = 128
HID_C = 8
OUT_C = 16

D = 16
NC = 2
NS = 16
NW = NC * NS
CHUNK = 128
CH = -(-E // (NW * CHUNK))
EPAD = NW * CH * CHUNK
NPAD = 10112
RPT = NPAD // NS
TPAD = 10016

_mesh = plsc.VectorSubcoreMesh(
    core_axis_name="c", subcore_axis_name="s", num_cores=NC, num_subcores=NS)
_sc_params = pltpu.CompilerParams(use_tc_tiling_on_sc=False)


def _zero_fill(buf, n_rows):
    def body(i, _):
        buf[i] = jnp.zeros((D,), jnp.float32)
        return 0
    lax.fori_loop(0, n_rows, body, 0)


@functools.partial(
    pl.kernel,
    out_type=jax.ShapeDtypeStruct((NC, NPAD, D), jnp.float32),
    mesh=_mesh,
    scratch_types=[
        pltpu.VMEM((CH, CHUNK), jnp.int32),
        pltpu.VMEM((CHUNK, D), jnp.float32),
        pltpu.VMEM((RPT, D), jnp.float32),
        pltpu.VMEM_SHARED((NPAD, D), jnp.float32),
    ],
    compiler_params=_sc_params,
)
def _sc_degree(dst_hbm, out_hbm, dst_v, ones_v, zbuf, acc_sh):
    c = lax.axis_index("c")
    s = lax.axis_index("s")
    _zero_fill(zbuf, RPT)

    def ones_fill(i, _):
        ones_v[i] = jnp.ones((D,), jnp.float32)
        return 0
    lax.fori_loop(0, CHUNK, ones_fill, 0)

    pltpu.sync_copy(zbuf, acc_sh.at[pl.ds(s * RPT, RPT)])
    plsc.subcore_barrier()
    pltpu.sync_copy(dst_hbm.at[c, s], dst_v)

    def body(j, _):
        pltpu.sync_copy(ones_v, acc_sh.at[dst_v.at[j]], add=True)
        return 0
    lax.fori_loop(0, CH, body, 0)

    plsc.subcore_barrier()
    pltpu.sync_copy(acc_sh.at[pl.ds(s * RPT, RPT)],
                    out_hbm.at[c].at[pl.ds(s * RPT, RPT)])


@functools.partial(
    pl.kernel,
    out_type=jax.ShapeDtypeStruct((NC, NPAD, D), jnp.float32),
    mesh=_mesh,
    scratch_types=[
        pltpu.VMEM((CH, CHUNK), jnp.int32),
        pltpu.VMEM((CH, CHUNK), jnp.int32),
        pltpu.VMEM((CHUNK, D), jnp.float32),
        pltpu.VMEM((RPT, D), jnp.float32),
        pltpu.VMEM_SHARED((NPAD, D), jnp.float32),
        pltpu.SemaphoreType.DMA,
    ],
    compiler_params=_sc_params,
)
def _sc_aggregate(src_hbm, dst_hbm, tab_hbm, out_hbm,
                  src_v, dst_v, rows_v, zbuf, acc_sh, sem):
    c = lax.axis_index("c")
    s = lax.axis_index("s")
    _zero_fill(zbuf, RPT)
    pltpu.sync_copy(zbuf, acc_sh.at[pl.ds(s * RPT, RPT)])
    plsc.subcore_barrier()
    pltpu.sync_copy(src_hbm.at[c, s], src_v)
    pltpu.sync_copy(dst_hbm.at[c, s], dst_v)

    def body(j, _):
        pltpu.async_copy(tab_hbm.at[src_v.at[j]], rows_v, sem).wait()
        pltpu.sync_copy(rows_v, acc_sh.at[dst_v.at[j]], add=True)
        return 0
    lax.fori_loop(0, CH, body, 0)

    plsc.subcore_barrier()
    pltpu.sync_copy(acc_sh.at[pl.ds(s * RPT, RPT)],
                    out_hbm.at[c].at[pl.ds(s * RPT, RPT)])


def _dinv_from(degp_ref):
    deg = degp_ref[0, :N, 0:1] + degp_ref[1, :N, 0:1] + 1.0
    return lax.rsqrt(deg)


def _tc_pre_body(x_ref, w_ref, degp_ref, h_ref, g_ref):
    h = jnp.dot(x_ref[...], w_ref[...], preferred_element_type=jnp.float32)
    dinv = _dinv_from(degp_ref)
    h_ref[...] = h
    g_ref[...] = h * dinv


def _tc_mid_body(acc_ref, degp_ref, h_ref, b_ref, z_ref, g_ref):
    dinv = _dinv_from(degp_ref)
    agg = acc_ref[0, :N, :HID_C] + acc_ref[1, :N, :HID_C]
    a = agg * dinv + (dinv * dinv) * h_ref[...] + b_ref[...]
    z = jnp.maximum(a, 0.0)
    z_ref[...] = z
    g_ref[...] = z * dinv


def _tc_post_body(acc_ref, degp_ref, z_ref, w_ref, b_ref, out_ref):
    dinv = _dinv_from(degp_ref)
    agg = acc_ref[0, :N, :HID_C] + acc_ref[1, :N, :HID_C]
    t = agg * dinv + (dinv * dinv) * z_ref[...]
    o = jnp.dot(t, w_ref[...], preferred_element_type=jnp.float32) + b_ref[...]
    m = jnp.max(o, axis=1, keepdims=True)
    e = jnp.exp(o - m)
    out_ref[...] = (o - m) - jnp.log(jnp.sum(e, axis=1, keepdims=True))


_f32 = jnp.float32


def kernel(x, edge_index, W1, b1, W2, b2):
    pad = jnp.full((EPAD - E,), N, dtype=jnp.int32)
    srcp = jnp.concatenate([edge_index[0], pad]).reshape(NC, NS, CH, CHUNK)
    dstp = jnp.concatenate([edge_index[1], pad]).reshape(NC, NS, CH, CHUNK)

    degp = _sc_degree(dstp)

    h, g1 = pl.pallas_call(
        _tc_pre_body,
        out_shape=(jax.ShapeDtypeStruct((N, HID_C), _f32),
                   jax.ShapeDtypeStruct((N, HID_C), _f32)),
    )(x, W1, degp)
    g1p = jnp.pad(g1, ((0, TPAD - N), (0, D - HID_C)))

    acc1 = _sc_aggregate(srcp, dstp, g1p)

    z, g2 = pl.pallas_call(
        _tc_mid_body,
        out_shape=(jax.ShapeDtypeStruct((N, HID_C), _f32),
                   jax.ShapeDtypeStruct((N, HID_C), _f32)),
    )(acc1, degp, h, b1.reshape(1, HID_C))
    g2p = jnp.pad(g2, ((0, TPAD - N), (0, D - HID_C)))

    acc2 = _sc_aggregate(srcp, dstp, g2p)

    out = pl.pallas_call(
        _tc_post_body,
        out_shape=jax.ShapeDtypeStruct((N, OUT_C), _f32),
    )(acc2, degp, z, W2, b2.reshape(1, OUT_C))
    return out


# speedup vs baseline: 37.9043x; 1.1634x over previous
import functools

import jax
import jax.numpy as jnp
from jax import lax
from jax.experimental import pallas as pl
from jax.experimental.pallas import tpu as pltpu
from jax.experimental.pallas import tpu_sc as plsc

N = 10000
E = 320000
IN_C = 128
HID_C = 8
OUT_C = 16

D = 16
NC = 2
NS = 16
NW = NC * NS
CHUNK = 128
NBUF = 8
CH = 80
WAVES = CH // NBUF
EPAD = NW * CH * CHUNK
NPAD = 10112
RPT = NPAD // NS
TPAD = 10016

_mesh = plsc.VectorSubcoreMesh(
    core_axis_name="c", subcore_axis_name="s", num_cores=NC, num_subcores=NS)
_sc_params = pltpu.CompilerParams(use_tc_tiling_on_sc=False)


def _zero_fill(buf, n_rows):
    def body(i, _):
        buf[i] = jnp.zeros((D,), jnp.float32)
        return 0
    lax.fori_loop(0, n_rows, body, 0)


def _drain(sem, dst, dummy_hbm_src):
    pltpu.make_async_copy(dummy_hbm_src, dst, sem).wait()


@functools.partial(
    pl.kernel,
    out_type=jax.ShapeDtypeStruct((NC, NPAD, D), jnp.float32),
    mesh=_mesh,
    scratch_types=[
        pltpu.VMEM((CH, CHUNK), jnp.int32),
        pltpu.VMEM((CHUNK, D), jnp.float32),
        pltpu.VMEM((RPT, D), jnp.float32),
        pltpu.VMEM_SHARED((NPAD, D), jnp.float32),
        pltpu.SemaphoreType.DMA,
    ],
    compiler_params=_sc_params,
)
def _sc_degree(dst_hbm, out_hbm, dst_v, ones_v, zbuf, acc_sh, sem):
    c = lax.axis_index("c")
    s = lax.axis_index("s")
    _zero_fill(zbuf, RPT)

    def ones_fill(i, _):
        ones_v[i] = jnp.ones((D,), jnp.float32)
        return 0
    lax.fori_loop(0, CHUNK, ones_fill, 0)

    pltpu.sync_copy(zbuf, acc_sh.at[pl.ds(s * RPT, RPT)])
    plsc.subcore_barrier()
    pltpu.sync_copy(dst_hbm.at[c, s], dst_v)

    def fire(j, _):
        pltpu.async_copy(ones_v, acc_sh.at[dst_v.at[j]], sem, add=True)
        return 0
    lax.fori_loop(0, CH, fire, 0)

    dummy = out_hbm.at[0].at[pl.ds(0, CHUNK)]

    def drain(j, _):
        _drain(sem, ones_v, dummy)
        return 0
    lax.fori_loop(0, CH, drain, 0)

    plsc.subcore_barrier()
    pltpu.sync_copy(acc_sh.at[pl.ds(s * RPT, RPT)],
                    out_hbm.at[c].at[pl.ds(s * RPT, RPT)])


@functools.partial(
    pl.kernel,
    out_type=jax.ShapeDtypeStruct((NC, NPAD, D), jnp.float32),
    mesh=_mesh,
    scratch_types=[
        pltpu.VMEM((CH, CHUNK), jnp.int32),
        pltpu.VMEM((CH, CHUNK), jnp.int32),
        pltpu.VMEM((NBUF, CHUNK, D), jnp.float32),
        pltpu.VMEM((RPT, D), jnp.float32),
        pltpu.VMEM_SHARED((NPAD, D), jnp.float32),
        pltpu.SemaphoreType.DMA((NBUF,)),
        pltpu.SemaphoreType.DMA((NBUF,)),
    ],
    compiler_params=_sc_params,
)
def _sc_aggregate(src_hbm, dst_hbm, tab_hbm, out_hbm,
                  src_v, dst_v, rows_v, zbuf, acc_sh, gsem, ssem):
    c = lax.axis_index("c")
    s = lax.axis_index("s")
    _zero_fill(zbuf, RPT)
    pltpu.sync_copy(zbuf, acc_sh.at[pl.ds(s * RPT, RPT)])
    plsc.subcore_barrier()
    pltpu.sync_copy(src_hbm.at[c, s], src_v)
    pltpu.sync_copy(dst_hbm.at[c, s], dst_v)

    dummy = tab_hbm.at[pl.ds(0, CHUNK)]

    for b in range(NBUF):
        pltpu.async_copy(tab_hbm.at[src_v.at[b]], rows_v.at[b], gsem.at[b])

    def wave(w, _):
        j0 = w * NBUF
        for b in range(NBUF):
            _drain(gsem.at[b], rows_v.at[b], dummy)
            pltpu.async_copy(rows_v.at[b], acc_sh.at[dst_v.at[j0 + b]],
                             ssem.at[b], add=True)

        @pl.when(w + 1 < WAVES)
        def _prefetch():
            for b in range(NBUF):
                _drain(ssem.at[b], rows_v.at[b], dummy)
                pltpu.async_copy(tab_hbm.at[src_v.at[j0 + NBUF + b]],
                                 rows_v.at[b], gsem.at[b])
        return 0
    lax.fori_loop(0, WAVES, wave, 0)

    for b in range(NBUF):
        _drain(ssem.at[b], rows_v.at[b], dummy)

    plsc.subcore_barrier()
    pltpu.sync_copy(acc_sh.at[pl.ds(s * RPT, RPT)],
                    out_hbm.at[c].at[pl.ds(s * RPT, RPT)])


def _dinv_from(degp_ref):
    deg = degp_ref[0, :N, 0:1] + degp_ref[1, :N, 0:1] + 1.0
    return lax.rsqrt(deg)


def _tc_pre_body(x_ref, w_ref, degp_ref, h_ref, g_ref):
    h = jnp.dot(x_ref[...], w_ref[...], preferred_element_type=jnp.float32)
    dinv = _dinv_from(degp_ref)
    h_ref[...] = h
    g_ref[...] = h * dinv


def _tc_mid_body(acc_ref, degp_ref, h_ref, b_ref, z_ref, g_ref):
    dinv = _dinv_from(degp_ref)
    agg = acc_ref[0, :N, :HID_C] + acc_ref[1, :N, :HID_C]
    a = agg * dinv + (dinv * dinv) * h_ref[...] + b_ref[...]
    z = jnp.maximum(a, 0.0)
    z_ref[...] = z
    g_ref[...] = z * dinv


def _tc_post_body(acc_ref, degp_ref, z_ref, w_ref, b_ref, out_ref):
    dinv = _dinv_from(degp_ref)
    agg = acc_ref[0, :N, :HID_C] + acc_ref[1, :N, :HID_C]
    t = agg * dinv + (dinv * dinv) * z_ref[...]
    o = jnp.dot(t, w_ref[...], preferred_element_type=jnp.float32) + b_ref[...]
    m = jnp.max(o, axis=1, keepdims=True)
    e = jnp.exp(o - m)
    out_ref[...] = (o - m) - jnp.log(jnp.sum(e, axis=1, keepdims=True))


_f32 = jnp.float32


def kernel(x, edge_index, W1, b1, W2, b2):
    pad = jnp.full((EPAD - E,), N, dtype=jnp.int32)
    srcp = jnp.concatenate([edge_index[0], pad]).reshape(NC, NS, CH, CHUNK)
    dstp = jnp.concatenate([edge_index[1], pad]).reshape(NC, NS, CH, CHUNK)

    degp = _sc_degree(dstp)

    h, g1 = pl.pallas_call(
        _tc_pre_body,
        out_shape=(jax.ShapeDtypeStruct((N, HID_C), _f32),
                   jax.ShapeDtypeStruct((N, HID_C), _f32)),
    )(x, W1, degp)
    g1p = jnp.pad(g1, ((0, TPAD - N), (0, D - HID_C)))

    acc1 = _sc_aggregate(srcp, dstp, g1p)

    z, g2 = pl.pallas_call(
        _tc_mid_body,
        out_shape=(jax.ShapeDtypeStruct((N, HID_C), _f32),
                   jax.ShapeDtypeStruct((N, HID_C), _f32)),
    )(acc1, degp, h, b1.reshape(1, HID_C))
    g2p = jnp.pad(g2, ((0, TPAD - N), (0, D - HID_C)))

    acc2 = _sc_aggregate(srcp, dstp, g2p)

    out = pl.pallas_call(
        _tc_post_body,
        out_shape=jax.ShapeDtypeStruct((N, OUT_C), _f32),
    )(acc2, degp, z, W2, b2.reshape(1, OUT_C))
    return out


# speedup vs baseline: 51.9393x; 1.3703x over previous
import functools

import jax
import jax.numpy as jnp
from jax import lax
from jax.experimental import pallas as pl
from jax.experimental.pallas import tpu as pltpu
from jax.experimental.pallas import tpu_sc as plsc

N = 10000
E = 320000
IN_C = 128
HID_C = 8
OUT_C = 16

D = 16
NC = 2
NS = 16
NW = NC * NS
CHUNK = 128
NBUF = 8
CH = 80
WAVES = CH // NBUF
EPAD = NW * CH * CHUNK
NPAD = 10112
RPT = NPAD // NS
TPAD = NPAD
TRPT = TPAD // NS

_mesh = plsc.VectorSubcoreMesh(
    core_axis_name="c", subcore_axis_name="s", num_cores=NC, num_subcores=NS)
_sc_params = pltpu.CompilerParams(use_tc_tiling_on_sc=False)


def _zero_fill(buf, n_rows):
    def body(i, _):
        buf[i] = jnp.zeros((D,), jnp.float32)
        return 0
    lax.fori_loop(0, n_rows, body, 0)


def _drain(sem, dst, dummy_hbm_src):
    pltpu.make_async_copy(dummy_hbm_src, dst, sem).wait()


@functools.partial(
    pl.kernel,
    out_type=jax.ShapeDtypeStruct((NC, NPAD, D), jnp.float32),
    mesh=_mesh,
    scratch_types=[
        pltpu.VMEM((CH, CHUNK), jnp.int32),
        pltpu.VMEM((CHUNK, D), jnp.float32),
        pltpu.VMEM((RPT, D), jnp.float32),
        pltpu.VMEM_SHARED((NPAD, D), jnp.float32),
        pltpu.SemaphoreType.DMA,
    ],
    compiler_params=_sc_params,
)
def _sc_degree(dst_hbm, out_hbm, dst_v, ones_v, zbuf, acc_sh, sem):
    c = lax.axis_index("c")
    s = lax.axis_index("s")
    _zero_fill(zbuf, RPT)

    def ones_fill(i, _):
        ones_v[i] = jnp.ones((D,), jnp.float32)
        return 0
    lax.fori_loop(0, CHUNK, ones_fill, 0)

    pltpu.sync_copy(zbuf, acc_sh.at[pl.ds(s * RPT, RPT)])
    plsc.subcore_barrier()
    pltpu.sync_copy(dst_hbm.at[c, s], dst_v)

    def fire(j, _):
        pltpu.async_copy(ones_v, acc_sh.at[dst_v.at[j]], sem, add=True)
        return 0
    lax.fori_loop(0, CH, fire, 0)

    dummy = out_hbm.at[0].at[pl.ds(0, CHUNK)]

    def drain(j, _):
        _drain(sem, ones_v, dummy)
        return 0
    lax.fori_loop(0, CH, drain, 0)

    plsc.subcore_barrier()
    pltpu.sync_copy(acc_sh.at[pl.ds(s * RPT, RPT)],
                    out_hbm.at[c].at[pl.ds(s * RPT, RPT)])


@functools.partial(
    pl.kernel,
    out_type=jax.ShapeDtypeStruct((NC, NPAD, D), jnp.float32),
    mesh=_mesh,
    scratch_types=[
        pltpu.VMEM((CH, CHUNK), jnp.int32),
        pltpu.VMEM((CH, CHUNK), jnp.int32),
        pltpu.VMEM((NBUF, CHUNK, D), jnp.float32),
        pltpu.VMEM((RPT, D), jnp.float32),
        pltpu.VMEM_SHARED((NPAD, D), jnp.float32),
        pltpu.VMEM_SHARED((TPAD, D), jnp.float32),
        pltpu.SemaphoreType.DMA((NBUF,)),
        pltpu.SemaphoreType.DMA((NBUF,)),
    ],
    compiler_params=_sc_params,
)
def _sc_aggregate(src_hbm, dst_hbm, tab_hbm, out_hbm,
                  src_v, dst_v, rows_v, zbuf, acc_sh, tab_sh, gsem, ssem):
    c = lax.axis_index("c")
    s = lax.axis_index("s")
    pltpu.sync_copy(tab_hbm.at[pl.ds(s * TRPT, TRPT)],
                    tab_sh.at[pl.ds(s * TRPT, TRPT)])
    _zero_fill(zbuf, RPT)
    pltpu.sync_copy(zbuf, acc_sh.at[pl.ds(s * RPT, RPT)])
    plsc.subcore_barrier()
    pltpu.sync_copy(src_hbm.at[c, s], src_v)
    pltpu.sync_copy(dst_hbm.at[c, s], dst_v)

    dummy = tab_hbm.at[pl.ds(0, CHUNK)]

    for b in range(NBUF):
        pltpu.async_copy(tab_sh.at[src_v.at[b]], rows_v.at[b], gsem.at[b])

    def wave(w, _):
        j0 = w * NBUF
        for b in range(NBUF):
            _drain(gsem.at[b], rows_v.at[b], dummy)
            pltpu.async_copy(rows_v.at[b], acc_sh.at[dst_v.at[j0 + b]],
                             ssem.at[b], add=True)

        @pl.when(w + 1 < WAVES)
        def _prefetch():
            for b in range(NBUF):
                _drain(ssem.at[b], rows_v.at[b], dummy)
                pltpu.async_copy(tab_sh.at[src_v.at[j0 + NBUF + b]],
                                 rows_v.at[b], gsem.at[b])
        return 0
    lax.fori_loop(0, WAVES, wave, 0)

    for b in range(NBUF):
        _drain(ssem.at[b], rows_v.at[b], dummy)

    plsc.subcore_barrier()
    pltpu.sync_copy(acc_sh.at[pl.ds(s * RPT, RPT)],
                    out_hbm.at[c].at[pl.ds(s * RPT, RPT)])


def _dinv_from(degp_ref):
    deg = degp_ref[0, :N, 0:1] + degp_ref[1, :N, 0:1] + 1.0
    return lax.rsqrt(deg)


def _tc_pre_body(x_ref, w_ref, degp_ref, h_ref, g_ref):
    h = jnp.dot(x_ref[...], w_ref[...], preferred_element_type=jnp.float32)
    dinv = _dinv_from(degp_ref)
    h_ref[...] = h
    g_ref[...] = h * dinv


def _tc_mid_body(acc_ref, degp_ref, h_ref, b_ref, z_ref, g_ref):
    dinv = _dinv_from(degp_ref)
    agg = acc_ref[0, :N, :HID_C] + acc_ref[1, :N, :HID_C]
    a = agg * dinv + (dinv * dinv) * h_ref[...] + b_ref[...]
    z = jnp.maximum(a, 0.0)
    z_ref[...] = z
    g_ref[...] = z * dinv


def _tc_post_body(acc_ref, degp_ref, z_ref, w_ref, b_ref, out_ref):
    dinv = _dinv_from(degp_ref)
    agg = acc_ref[0, :N, :HID_C] + acc_ref[1, :N, :HID_C]
    t = agg * dinv + (dinv * dinv) * z_ref[...]
    o = jnp.dot(t, w_ref[...], preferred_element_type=jnp.float32) + b_ref[...]
    m = jnp.max(o, axis=1, keepdims=True)
    e = jnp.exp(o - m)
    out_ref[...] = (o - m) - jnp.log(jnp.sum(e, axis=1, keepdims=True))


_f32 = jnp.float32


def kernel(x, edge_index, W1, b1, W2, b2):
    pad = jnp.full((EPAD - E,), N, dtype=jnp.int32)
    srcp = jnp.concatenate([edge_index[0], pad]).reshape(NC, NS, CH, CHUNK)
    dstp = jnp.concatenate([edge_index[1], pad]).reshape(NC, NS, CH, CHUNK)

    degp = _sc_degree(dstp)

    h, g1 = pl.pallas_call(
        _tc_pre_body,
        out_shape=(jax.ShapeDtypeStruct((N, HID_C), _f32),
                   jax.ShapeDtypeStruct((N, HID_C), _f32)),
    )(x, W1, degp)
    g1p = jnp.pad(g1, ((0, TPAD - N), (0, D - HID_C)))

    acc1 = _sc_aggregate(srcp, dstp, g1p)

    z, g2 = pl.pallas_call(
        _tc_mid_body,
        out_shape=(jax.ShapeDtypeStruct((N, HID_C), _f32),
                   jax.ShapeDtypeStruct((N, HID_C), _f32)),
    )(acc1, degp, h, b1.reshape(1, HID_C))
    g2p = jnp.pad(g2, ((0, TPAD - N), (0, D - HID_C)))

    acc2 = _sc_aggregate(srcp, dstp, g2p)

    out = pl.pallas_call(
        _tc_post_body,
        out_shape=jax.ShapeDtypeStruct((N, OUT_C), _f32),
    )(acc2, degp, z, W2, b2.reshape(1, OUT_C))
    return out


# speedup vs baseline: 56.2381x; 1.0828x over previous
import functools

import jax
import jax.numpy as jnp
from jax import lax
from jax.experimental import pallas as pl
from jax.experimental.pallas import tpu as pltpu
from jax.experimental.pallas import tpu_sc as plsc

N = 10000
E = 320000
IN_C = 128
HID_C = 8
OUT_C = 16

D = 8
NC = 2
NS = 16
NW = NC * NS
CHUNK = 128
NBUF = 8
CH = 80
WAVES = CH // NBUF
EPAD = NW * CH * CHUNK
NPAD = 10112
RPT = NPAD // NS
TPAD = NPAD
TRPT = TPAD // NS

_mesh = plsc.VectorSubcoreMesh(
    core_axis_name="c", subcore_axis_name="s", num_cores=NC, num_subcores=NS)
_sc_params = pltpu.CompilerParams(use_tc_tiling_on_sc=False)


def _zero_fill(buf, n_rows):
    def body(i, _):
        buf[i] = jnp.zeros((D,), jnp.float32)
        return 0
    lax.fori_loop(0, n_rows, body, 0)


def _drain(sem, dst, dummy_hbm_src):
    pltpu.make_async_copy(dummy_hbm_src, dst, sem).wait()


@functools.partial(
    pl.kernel,
    out_type=jax.ShapeDtypeStruct((NC, NPAD, D), jnp.float32),
    mesh=_mesh,
    scratch_types=[
        pltpu.VMEM((CH, CHUNK), jnp.int32),
        pltpu.VMEM((CHUNK, D), jnp.float32),
        pltpu.VMEM((RPT, D), jnp.float32),
        pltpu.VMEM_SHARED((NPAD, D), jnp.float32),
        pltpu.SemaphoreType.DMA,
    ],
    compiler_params=_sc_params,
)
def _sc_degree(dst_hbm, out_hbm, dst_v, ones_v, zbuf, acc_sh, sem):
    c = lax.axis_index("c")
    s = lax.axis_index("s")
    _zero_fill(zbuf, RPT)

    def ones_fill(i, _):
        ones_v[i] = jnp.ones((D,), jnp.float32)
        return 0
    lax.fori_loop(0, CHUNK, ones_fill, 0)

    pltpu.sync_copy(zbuf, acc_sh.at[pl.ds(s * RPT, RPT)])
    plsc.subcore_barrier()
    pltpu.sync_copy(dst_hbm.at[c, s], dst_v)

    def fire(j, _):
        pltpu.async_copy(ones_v, acc_sh.at[dst_v.at[j]], sem, add=True)
        return 0
    lax.fori_loop(0, CH, fire, 0)

    dummy = out_hbm.at[0].at[pl.ds(0, CHUNK)]

    def drain(j, _):
        _drain(sem, ones_v, dummy)
        return 0
    lax.fori_loop(0, CH, drain, 0)

    plsc.subcore_barrier()
    pltpu.sync_copy(acc_sh.at[pl.ds(s * RPT, RPT)],
                    out_hbm.at[c].at[pl.ds(s * RPT, RPT)])


@functools.partial(
    pl.kernel,
    out_type=jax.ShapeDtypeStruct((NC, NPAD, D), jnp.float32),
    mesh=_mesh,
    scratch_types=[
        pltpu.VMEM((CH, CHUNK), jnp.int32),
        pltpu.VMEM((CH, CHUNK), jnp.int32),
        pltpu.VMEM((NBUF, CHUNK, D), jnp.float32),
        pltpu.VMEM((RPT, D), jnp.float32),
        pltpu.VMEM_SHARED((NPAD, D), jnp.float32),
        pltpu.VMEM_SHARED((TPAD, D), jnp.float32),
        pltpu.SemaphoreType.DMA((NBUF,)),
        pltpu.SemaphoreType.DMA((NBUF,)),
    ],
    compiler_params=_sc_params,
)
def _sc_aggregate(src_hbm, dst_hbm, tab_hbm, out_hbm,
                  src_v, dst_v, rows_v, zbuf, acc_sh, tab_sh, gsem, ssem):
    c = lax.axis_index("c")
    s = lax.axis_index("s")
    pltpu.sync_copy(tab_hbm.at[pl.ds(s * TRPT, TRPT)],
                    tab_sh.at[pl.ds(s * TRPT, TRPT)])
    _zero_fill(zbuf, RPT)
    pltpu.sync_copy(zbuf, acc_sh.at[pl.ds(s * RPT, RPT)])
    plsc.subcore_barrier()
    pltpu.sync_copy(src_hbm.at[c, s], src_v)
    pltpu.sync_copy(dst_hbm.at[c, s], dst_v)

    dummy = tab_hbm.at[pl.ds(0, CHUNK)]

    for b in range(NBUF):
        pltpu.async_copy(tab_sh.at[src_v.at[b]], rows_v.at[b], gsem.at[b])

    def wave(w, _):
        j0 = w * NBUF
        for b in range(NBUF):
            _drain(gsem.at[b], rows_v.at[b], dummy)
            pltpu.async_copy(rows_v.at[b], acc_sh.at[dst_v.at[j0 + b]],
                             ssem.at[b], add=True)

        @pl.when(w + 1 < WAVES)
        def _prefetch():
            for b in range(NBUF):
                _drain(ssem.at[b], rows_v.at[b], dummy)
                pltpu.async_copy(tab_sh.at[src_v.at[j0 + NBUF + b]],
                                 rows_v.at[b], gsem.at[b])
        return 0
    lax.fori_loop(0, WAVES, wave, 0)

    for b in range(NBUF):
        _drain(ssem.at[b], rows_v.at[b], dummy)

    plsc.subcore_barrier()
    pltpu.sync_copy(acc_sh.at[pl.ds(s * RPT, RPT)],
                    out_hbm.at[c].at[pl.ds(s * RPT, RPT)])


def _dinv_from(degp_ref):
    deg = degp_ref[0, :N, 0:1] + degp_ref[1, :N, 0:1] + 1.0
    return lax.rsqrt(deg)


def _tc_pre_body(x_ref, w_ref, degp_ref, h_ref, g_ref):
    h = jnp.dot(x_ref[...], w_ref[...], preferred_element_type=jnp.float32)
    dinv = _dinv_from(degp_ref)
    h_ref[...] = h
    g_ref[...] = h * dinv


def _tc_mid_body(acc_ref, degp_ref, h_ref, b_ref, z_ref, g_ref):
    dinv = _dinv_from(degp_ref)
    agg = acc_ref[0, :N, :HID_C] + acc_ref[1, :N, :HID_C]
    a = agg * dinv + (dinv * dinv) * h_ref[...] + b_ref[...]
    z = jnp.maximum(a, 0.0)
    z_ref[...] = z
    g_ref[...] = z * dinv


def _tc_post_body(acc_ref, degp_ref, z_ref, w_ref, b_ref, out_ref):
    dinv = _dinv_from(degp_ref)
    agg = acc_ref[0, :N, :HID_C] + acc_ref[1, :N, :HID_C]
    t = agg * dinv + (dinv * dinv) * z_ref[...]
    o = jnp.dot(t, w_ref[...], preferred_element_type=jnp.float32) + b_ref[...]
    m = jnp.max(o, axis=1, keepdims=True)
    e = jnp.exp(o - m)
    out_ref[...] = (o - m) - jnp.log(jnp.sum(e, axis=1, keepdims=True))


_f32 = jnp.float32


def kernel(x, edge_index, W1, b1, W2, b2):
    pad = jnp.full((EPAD - E,), N, dtype=jnp.int32)
    srcp = jnp.concatenate([edge_index[0], pad]).reshape(NC, NS, CH, CHUNK)
    dstp = jnp.concatenate([edge_index[1], pad]).reshape(NC, NS, CH, CHUNK)

    degp = _sc_degree(dstp)

    h, g1 = pl.pallas_call(
        _tc_pre_body,
        out_shape=(jax.ShapeDtypeStruct((N, HID_C), _f32),
                   jax.ShapeDtypeStruct((N, HID_C), _f32)),
    )(x, W1, degp)
    g1p = jnp.pad(g1, ((0, TPAD - N), (0, D - HID_C)))

    acc1 = _sc_aggregate(srcp, dstp, g1p)

    z, g2 = pl.pallas_call(
        _tc_mid_body,
        out_shape=(jax.ShapeDtypeStruct((N, HID_C), _f32),
                   jax.ShapeDtypeStruct((N, HID_C), _f32)),
    )(acc1, degp, h, b1.reshape(1, HID_C))
    g2p = jnp.pad(g2, ((0, TPAD - N), (0, D - HID_C)))

    acc2 = _sc_aggregate(srcp, dstp, g2p)

    out = pl.pallas_call(
        _tc_post_body,
        out_shape=jax.ShapeDtypeStruct((N, OUT_C), _f32),
    )(acc2, degp, z, W2, b2.reshape(1, OUT_C))
    return out


# speedup vs baseline: 63.3119x; 1.1258x over previous
import functools

import jax
import jax.numpy as jnp
from jax import lax
from jax.experimental import pallas as pl
from jax.experimental.pallas import tpu as pltpu
from jax.experimental.pallas import tpu_sc as plsc

N = 10000
E = 320000
IN_C = 128
HID_C = 8
OUT_C = 16

D = 8
NC = 2
NS = 16
NW = NC * NS
CHUNK = 125
CH = 80
NBUF = 8
WAVES = CH // NBUF
NPAD = 10112
RPT = NPAD // NS
TPAD = NPAD
TRPT = TPAD // NS

_mesh = plsc.VectorSubcoreMesh(
    core_axis_name="c", subcore_axis_name="s", num_cores=NC, num_subcores=NS)
_sc_params = pltpu.CompilerParams(use_tc_tiling_on_sc=False)


def _zero_fill(buf, n_rows):
    def body(i, _):
        buf[i] = jnp.zeros((D,), jnp.float32)
        return 0
    lax.fori_loop(0, n_rows, body, 0)


def _drain(sem, dst, dummy_hbm_src):
    pltpu.make_async_copy(dummy_hbm_src, dst, sem).wait()


@functools.partial(
    pl.kernel,
    out_type=jax.ShapeDtypeStruct((NC, NPAD, D), jnp.float32),
    mesh=_mesh,
    scratch_types=[
        pltpu.VMEM((CH, CHUNK), jnp.int32),
        pltpu.VMEM((CHUNK, D), jnp.float32),
        pltpu.VMEM((RPT, D), jnp.float32),
        pltpu.VMEM_SHARED((NPAD, D), jnp.float32),
        pltpu.SemaphoreType.DMA,
    ],
    compiler_params=_sc_params,
)
def _sc_degree(dst_hbm, out_hbm, dst_v, ones_v, zbuf, acc_sh, sem):
    c = lax.axis_index("c")
    s = lax.axis_index("s")
    _zero_fill(zbuf, RPT)

    def ones_fill(i, _):
        ones_v[i] = jnp.ones((D,), jnp.float32)
        return 0
    lax.fori_loop(0, CHUNK, ones_fill, 0)

    pltpu.sync_copy(zbuf, acc_sh.at[pl.ds(s * RPT, RPT)])
    plsc.subcore_barrier()
    pltpu.sync_copy(dst_hbm.at[c, s], dst_v)

    def fire(j, _):
        pltpu.async_copy(ones_v, acc_sh.at[dst_v.at[j]], sem, add=True)
        return 0
    lax.fori_loop(0, CH, fire, 0)

    dummy = out_hbm.at[0].at[pl.ds(0, CHUNK)]

    def drain(j, _):
        _drain(sem, ones_v, dummy)
        return 0
    lax.fori_loop(0, CH, drain, 0)

    plsc.subcore_barrier()
    pltpu.sync_copy(acc_sh.at[pl.ds(s * RPT, RPT)],
                    out_hbm.at[c].at[pl.ds(s * RPT, RPT)])


@functools.partial(
    pl.kernel,
    out_type=jax.ShapeDtypeStruct((NC, NPAD, D), jnp.float32),
    mesh=_mesh,
    scratch_types=[
        pltpu.VMEM((CH, CHUNK), jnp.int32),
        pltpu.VMEM((CH, CHUNK), jnp.int32),
        pltpu.VMEM((NBUF, CHUNK, D), jnp.float32),
        pltpu.VMEM((RPT, D), jnp.float32),
        pltpu.VMEM_SHARED((NPAD, D), jnp.float32),
        pltpu.VMEM_SHARED((TPAD, D), jnp.float32),
        pltpu.SemaphoreType.DMA((NBUF,)),
        pltpu.SemaphoreType.DMA((NBUF,)),
    ],
    compiler_params=_sc_params,
)
def _sc_aggregate(src_hbm, dst_hbm, tab_hbm, out_hbm,
                  src_v, dst_v, rows_v, zbuf, acc_sh, tab_sh, gsem, ssem):
    c = lax.axis_index("c")
    s = lax.axis_index("s")
    pltpu.sync_copy(tab_hbm.at[pl.ds(s * TRPT, TRPT)],
                    tab_sh.at[pl.ds(s * TRPT, TRPT)])
    _zero_fill(zbuf, RPT)
    pltpu.sync_copy(zbuf, acc_sh.at[pl.ds(s * RPT, RPT)])
    plsc.subcore_barrier()
    pltpu.sync_copy(src_hbm.at[c, s], src_v)
    pltpu.sync_copy(dst_hbm.at[c, s], dst_v)

    dummy = tab_hbm.at[pl.ds(0, CHUNK)]

    for b in range(NBUF):
        pltpu.async_copy(tab_sh.at[src_v.at[b]], rows_v.at[b], gsem.at[b])

    def wave(w, _):
        j0 = w * NBUF
        for b in range(NBUF):
            _drain(gsem.at[b], rows_v.at[b], dummy)
            pltpu.async_copy(rows_v.at[b], acc_sh.at[dst_v.at[j0 + b]],
                             ssem.at[b], add=True)

        @pl.when(w + 1 < WAVES)
        def _prefetch():
            for b in range(NBUF):
                _drain(ssem.at[b], rows_v.at[b], dummy)
                pltpu.async_copy(tab_sh.at[src_v.at[j0 + NBUF + b]],
                                 rows_v.at[b], gsem.at[b])
        return 0
    lax.fori_loop(0, WAVES, wave, 0)

    for b in range(NBUF):
        _drain(ssem.at[b], rows_v.at[b], dummy)

    plsc.subcore_barrier()
    pltpu.sync_copy(acc_sh.at[pl.ds(s * RPT, RPT)],
                    out_hbm.at[c].at[pl.ds(s * RPT, RPT)])


def _tc_mm_body(x_ref, w_ref, h_ref):
    h_ref[...] = jnp.dot(x_ref[...], w_ref[...],
                         preferred_element_type=jnp.float32)


def _tc_scale_body(degp_ref, h_ref, dinv_ref, g_ref):
    deg = degp_ref[0, :N, 0:1] + degp_ref[1, :N, 0:1] + 1.0
    dinv = lax.rsqrt(deg)
    dinv_ref[...] = dinv
    g_ref[...] = jnp.pad(h_ref[...] * dinv, ((0, TPAD - N), (0, 0)))


def _tc_mid_body(acc_ref, dinv_ref, h_ref, b_ref, z_ref, g_ref):
    dinv = dinv_ref[...]
    agg = acc_ref[0, :N, :] + acc_ref[1, :N, :]
    a = agg * dinv + (dinv * dinv) * h_ref[...] + b_ref[...]
    z = jnp.maximum(a, 0.0)
    z_ref[...] = z
    g_ref[...] = jnp.pad(z * dinv, ((0, TPAD - N), (0, 0)))


def _tc_post_body(acc_ref, dinv_ref, z_ref, w_ref, b_ref, out_ref):
    dinv = dinv_ref[...]
    agg = acc_ref[0, :N, :] + acc_ref[1, :N, :]
    t = agg * dinv + (dinv * dinv) * z_ref[...]
    o = jnp.dot(t, w_ref[...], preferred_element_type=jnp.float32) + b_ref[...]
    m = jnp.max(o, axis=1, keepdims=True)
    e = jnp.exp(o - m)
    out_ref[...] = (o - m) - jnp.log(jnp.sum(e, axis=1, keepdims=True))


_f32 = jnp.float32


def kernel(x, edge_index, W1, b1, W2, b2):
    srcp = edge_index[0].reshape(NC, NS, CH, CHUNK)
    dstp = edge_index[1].reshape(NC, NS, CH, CHUNK)

    degp = _sc_degree(dstp)
    h = pl.pallas_call(
        _tc_mm_body,
        out_shape=jax.ShapeDtypeStruct((N, HID_C), _f32),
    )(x, W1)

    dinv, g1p = pl.pallas_call(
        _tc_scale_body,
        out_shape=(jax.ShapeDtypeStruct((N, 1), _f32),
                   jax.ShapeDtypeStruct((TPAD, D), _f32)),
    )(degp, h)

    acc1 = _sc_aggregate(srcp, dstp, g1p)

    z, g2p = pl.pallas_call(
        _tc_mid_body,
        out_shape=(jax.ShapeDtypeStruct((N, HID_C), _f32),
                   jax.ShapeDtypeStruct((TPAD, D), _f32)),
    )(acc1, dinv, h, b1.reshape(1, HID_C))

    acc2 = _sc_aggregate(srcp, dstp, g2p)

    out = pl.pallas_call(
        _tc_post_body,
        out_shape=jax.ShapeDtypeStruct((N, OUT_C), _f32),
    )(acc2, dinv, z, W2, b2.reshape(1, OUT_C))
    return out


# speedup vs baseline: 95.0924x; 1.5020x over previous
import functools

import jax
import jax.numpy as jnp
from jax import lax
from jax.experimental import pallas as pl
from jax.experimental.pallas import tpu as pltpu
from jax.experimental.pallas import tpu_sc as plsc

N = 10000
E = 320000
IN_C = 128
HID_C = 8
OUT_C = 16

D = 8
NC = 2
NS = 16
NW = NC * NS
CHUNK = 125
CH = 80
NBUF = 8
WAVES = CH // NBUF
NPAD = 10240
RPT = NPAD // NS
PK = 128
PR = NPAD * D // PK
PRPT = PR // NS
NROWS = N * D // PK

_mesh = plsc.VectorSubcoreMesh(
    core_axis_name="c", subcore_axis_name="s", num_cores=NC, num_subcores=NS)
_sc_params = pltpu.CompilerParams(use_tc_tiling_on_sc=False)


def _zero_fill(buf, n_rows):
    def body(i, _):
        buf[i] = jnp.zeros((D,), jnp.float32)
        return 0
    lax.fori_loop(0, n_rows, body, 0)


def _drain(sem, dst, dummy_hbm_src):
    pltpu.make_async_copy(dummy_hbm_src, dst, sem).wait()


@functools.partial(
    pl.kernel,
    out_type=jax.ShapeDtypeStruct((NC, NPAD, D), jnp.float32),
    mesh=_mesh,
    scratch_types=[
        pltpu.VMEM((CH, CHUNK), jnp.int32),
        pltpu.VMEM((CHUNK, D), jnp.float32),
        pltpu.VMEM((RPT, D), jnp.float32),
        pltpu.VMEM_SHARED((NPAD, D), jnp.float32),
        pltpu.SemaphoreType.DMA,
    ],
    compiler_params=_sc_params,
)
def _sc_degree(dst_hbm, out_hbm, dst_v, ones_v, zbuf, acc_sh, sem):
    c = lax.axis_index("c")
    s = lax.axis_index("s")
    _zero_fill(zbuf, RPT)

    def ones_fill(i, _):
        ones_v[i] = jnp.ones((D,), jnp.float32)
        return 0
    lax.fori_loop(0, CHUNK, ones_fill, 0)

    pltpu.sync_copy(zbuf, acc_sh.at[pl.ds(s * RPT, RPT)])
    plsc.subcore_barrier()
    pltpu.sync_copy(dst_hbm.at[c, s], dst_v)

    def fire(j, _):
        pltpu.async_copy(ones_v, acc_sh.at[dst_v.at[j]], sem, add=True)
        return 0
    lax.fori_loop(0, CH, fire, 0)

    dummy = out_hbm.at[0].at[pl.ds(0, CHUNK)]

    def drain(j, _):
        _drain(sem, ones_v, dummy)
        return 0
    lax.fori_loop(0, CH, drain, 0)

    plsc.subcore_barrier()
    pltpu.sync_copy(acc_sh.at[pl.ds(s * RPT, RPT)],
                    out_hbm.at[c].at[pl.ds(s * RPT, RPT)])


@functools.partial(
    pl.kernel,
    out_type=jax.ShapeDtypeStruct((NC, NPAD, D), jnp.float32),
    mesh=_mesh,
    scratch_types=[
        pltpu.VMEM((CH, CHUNK), jnp.int32),
        pltpu.VMEM((CH, CHUNK), jnp.int32),
        pltpu.VMEM((NBUF, CHUNK, D), jnp.float32),
        pltpu.VMEM((RPT, D), jnp.float32),
        pltpu.VMEM_SHARED((NPAD, D), jnp.float32),
        pltpu.VMEM_SHARED((NPAD, D), jnp.float32),
        pltpu.SemaphoreType.DMA((NBUF,)),
        pltpu.SemaphoreType.DMA((NBUF,)),
    ],
    compiler_params=_sc_params,
)
def _sc_aggregate(src_hbm, dst_hbm, tab_hbm, out_hbm,
                  src_v, dst_v, rows_v, zbuf, acc_sh, tab_sh, gsem, ssem):
    c = lax.axis_index("c")
    s = lax.axis_index("s")
    pltpu.sync_copy(tab_hbm.at[pl.ds(s * RPT, RPT)],
                    tab_sh.at[pl.ds(s * RPT, RPT)])
    _zero_fill(zbuf, RPT)
    pltpu.sync_copy(zbuf, acc_sh.at[pl.ds(s * RPT, RPT)])
    plsc.subcore_barrier()
    pltpu.sync_copy(src_hbm.at[c, s], src_v)
    pltpu.sync_copy(dst_hbm.at[c, s], dst_v)

    dummy = tab_hbm.at[pl.ds(0, CHUNK)]

    for b in range(NBUF):
        pltpu.async_copy(tab_sh.at[src_v.at[b]], rows_v.at[b], gsem.at[b])

    def wave(w, _):
        j0 = w * NBUF
        for b in range(NBUF):
            _drain(gsem.at[b], rows_v.at[b], dummy)
            pltpu.async_copy(rows_v.at[b], acc_sh.at[dst_v.at[j0 + b]],
                             ssem.at[b], add=True)

        @pl.when(w + 1 < WAVES)
        def _prefetch():
            for b in range(NBUF):
                _drain(ssem.at[b], rows_v.at[b], dummy)
                pltpu.async_copy(tab_sh.at[src_v.at[j0 + NBUF + b]],
                                 rows_v.at[b], gsem.at[b])
        return 0
    lax.fori_loop(0, WAVES, wave, 0)

    for b in range(NBUF):
        _drain(ssem.at[b], rows_v.at[b], dummy)

    plsc.subcore_barrier()
    pltpu.sync_copy(acc_sh.at[pl.ds(s * RPT, RPT)],
                    out_hbm.at[c].at[pl.ds(s * RPT, RPT)])


def _tc_mm_body(x2_ref, w1k_ref, h_ref):
    h_ref[...] = jnp.dot(x2_ref[...], w1k_ref[...],
                         preferred_element_type=jnp.float32)


def _tc_scale_body(degp_ref, h_ref, dinv_ref, g_ref):
    deg = degp_ref[0] + degp_ref[1] + 1.0
    dinv = lax.rsqrt(deg)
    dinv_ref[...] = dinv
    hp = jnp.pad(h_ref[...], ((0, PR - NROWS), (0, 0)))
    g_ref[...] = hp * dinv


def _tc_mid_body(acc_ref, dinv_ref, h_ref, b_ref, z_ref, g_ref):
    dinv = dinv_ref[...]
    hp = jnp.pad(h_ref[...], ((0, PR - NROWS), (0, 0)))
    a = (acc_ref[0] + acc_ref[1]) * dinv + (dinv * dinv) * hp + b_ref[...]
    z = jnp.maximum(a, 0.0)
    z_ref[...] = z
    g_ref[...] = z * dinv


def _tc_post_body(acc_ref, dinv_ref, z_ref, w2k_ref, b_ref, out_ref):
    dinv = dinv_ref[...]
    t = (acc_ref[0] + acc_ref[1]) * dinv + (dinv * dinv) * z_ref[...]
    o = jnp.dot(t, w2k_ref[...], preferred_element_type=jnp.float32)
    o = o + b_ref[...]
    pieces = []
    for b in range(16):
        ob = o[:, b * OUT_C:(b + 1) * OUT_C]
        m = jnp.max(ob, axis=1, keepdims=True)
        e = jnp.exp(ob - m)
        pieces.append((ob - m) - jnp.log(jnp.sum(e, axis=1, keepdims=True)))
    out_ref[...] = jnp.concatenate(pieces, axis=1)


_f32 = jnp.float32


def kernel(x, edge_index, W1, b1, W2, b2):
    srcp = edge_index[0].reshape(NC, NS, CH, CHUNK)
    dstp = edge_index[1].reshape(NC, NS, CH, CHUNK)

    w1k = jnp.kron(jnp.eye(16, dtype=_f32), W1)
    w2k = jnp.kron(jnp.eye(16, dtype=_f32), W2)
    b1k = jnp.tile(b1, 16).reshape(1, PK)
    b2k = jnp.tile(b2, 16).reshape(1, 16 * OUT_C)
    x2 = x.reshape(N // 16, 16 * IN_C)

    degp = _sc_degree(dstp)
    h = pl.pallas_call(
        _tc_mm_body,
        out_shape=jax.ShapeDtypeStruct((NROWS, PK), _f32),
    )(x2, w1k)

    dinv, g1p = pl.pallas_call(
        _tc_scale_body,
        out_shape=(jax.ShapeDtypeStruct((PR, PK), _f32),
                   jax.ShapeDtypeStruct((PR, PK), _f32)),
    )(degp.reshape(NC, PR, PK), h)

    acc1 = _sc_aggregate(srcp, dstp, g1p.reshape(NPAD, D))

    z, g2p = pl.pallas_call(
        _tc_mid_body,
        out_shape=(jax.ShapeDtypeStruct((PR, PK), _f32),
                   jax.ShapeDtypeStruct((PR, PK), _f32)),
    )(acc1.reshape(NC, PR, PK), dinv, h, b1k)

    acc2 = _sc_aggregate(srcp, dstp, g2p.reshape(NPAD, D))

    op = pl.pallas_call(
        _tc_post_body,
        out_shape=jax.ShapeDtypeStruct((PR, 16 * OUT_C), _f32),
    )(acc2.reshape(NC, PR, PK), dinv, z, w2k, b2k)
    return op.reshape(NPAD, OUT_C)[:N]


# speedup vs baseline: 99.8494x; 1.0500x over previous
import functools

import jax
import jax.numpy as jnp
from jax import lax
from jax.experimental import pallas as pl
from jax.experimental.pallas import tpu as pltpu
from jax.experimental.pallas import tpu_sc as plsc

N = 10000
E = 320000
IN_C = 128
HID_C = 8
OUT_C = 16

D = 8
NC = 2
NS = 16
NW = NC * NS
CHUNK = 125
CH = 80
NBUF = 10
WAVES = CH // NBUF
NPAD = 10240
RPT = NPAD // NS
PK = 128
PR = NPAD * D // PK
PRPT = PR // NS
NROWS = N * D // PK

_mesh = plsc.VectorSubcoreMesh(
    core_axis_name="c", subcore_axis_name="s", num_cores=NC, num_subcores=NS)
_sc_params = pltpu.CompilerParams(use_tc_tiling_on_sc=False)


def _zero_fill(buf, n_rows):
    def body(i, _):
        buf[i] = jnp.zeros((D,), jnp.float32)
        return 0
    lax.fori_loop(0, n_rows, body, 0)


def _drain(sem, dst, dummy_hbm_src):
    pltpu.make_async_copy(dummy_hbm_src, dst, sem).wait()


@functools.partial(
    pl.kernel,
    out_type=jax.ShapeDtypeStruct((NC, NPAD, D), jnp.float32),
    mesh=_mesh,
    scratch_types=[
        pltpu.VMEM((CH, CHUNK), jnp.int32),
        pltpu.VMEM((CHUNK, D), jnp.float32),
        pltpu.VMEM((RPT, D), jnp.float32),
        pltpu.VMEM_SHARED((NPAD, D), jnp.float32),
        pltpu.SemaphoreType.DMA,
        pltpu.SemaphoreType.DMA,
    ],
    compiler_params=_sc_params,
)
def _sc_degree(dst_hbm, out_hbm, dst_v, ones_v, zbuf, acc_sh, sem, lsem):
    c = lax.axis_index("c")
    s = lax.axis_index("s")
    cp_dst = pltpu.async_copy(dst_hbm.at[c, s], dst_v, lsem)
    _zero_fill(zbuf, RPT)

    def ones_fill(i, _):
        ones_v[i] = jnp.ones((D,), jnp.float32)
        return 0
    lax.fori_loop(0, CHUNK, ones_fill, 0)

    pltpu.sync_copy(zbuf, acc_sh.at[pl.ds(s * RPT, RPT)])
    cp_dst.wait()
    plsc.subcore_barrier()

    def fire(j, _):
        pltpu.async_copy(ones_v, acc_sh.at[dst_v.at[j]], sem, add=True)
        return 0
    lax.fori_loop(0, CH, fire, 0)

    dummy = out_hbm.at[0].at[pl.ds(0, CHUNK)]

    def drain(j, _):
        _drain(sem, ones_v, dummy)
        return 0
    lax.fori_loop(0, CH, drain, 0)

    plsc.subcore_barrier()
    pltpu.sync_copy(acc_sh.at[pl.ds(s * RPT, RPT)],
                    out_hbm.at[c].at[pl.ds(s * RPT, RPT)])


@functools.partial(
    pl.kernel,
    out_type=jax.ShapeDtypeStruct((NC, NPAD, D), jnp.float32),
    mesh=_mesh,
    scratch_types=[
        pltpu.VMEM((CH, CHUNK), jnp.int32),
        pltpu.VMEM((CH, CHUNK), jnp.int32),
        pltpu.VMEM((NBUF, CHUNK, D), jnp.float32),
        pltpu.VMEM((RPT, D), jnp.float32),
        pltpu.VMEM_SHARED((NPAD, D), jnp.float32),
        pltpu.VMEM_SHARED((NPAD, D), jnp.float32),
        pltpu.SemaphoreType.DMA((NBUF,)),
        pltpu.SemaphoreType.DMA((NBUF,)),
        pltpu.SemaphoreType.DMA,
    ],
    compiler_params=_sc_params,
)
def _sc_aggregate(src_hbm, dst_hbm, tab_hbm, out_hbm,
                  src_v, dst_v, rows_v, zbuf, acc_sh, tab_sh, gsem, ssem,
                  lsem):
    c = lax.axis_index("c")
    s = lax.axis_index("s")
    cp_src = pltpu.async_copy(src_hbm.at[c, s], src_v, lsem)
    cp_dst = pltpu.async_copy(dst_hbm.at[c, s], dst_v, lsem)
    pltpu.sync_copy(tab_hbm.at[pl.ds(s * RPT, RPT)],
                    tab_sh.at[pl.ds(s * RPT, RPT)])
    _zero_fill(zbuf, RPT)
    pltpu.sync_copy(zbuf, acc_sh.at[pl.ds(s * RPT, RPT)])
    cp_src.wait()
    cp_dst.wait()
    plsc.subcore_barrier()

    dummy = tab_hbm.at[pl.ds(0, CHUNK)]

    for b in range(NBUF):
        pltpu.async_copy(tab_sh.at[src_v.at[b]], rows_v.at[b], gsem.at[b])

    def wave(w, _):
        j0 = w * NBUF
        for b in range(NBUF):
            _drain(gsem.at[b], rows_v.at[b], dummy)
            pltpu.async_copy(rows_v.at[b], acc_sh.at[dst_v.at[j0 + b]],
                             ssem.at[b], add=True)

        @pl.when(w + 1 < WAVES)
        def _prefetch():
            for b in range(NBUF):
                _drain(ssem.at[b], rows_v.at[b], dummy)
                pltpu.async_copy(tab_sh.at[src_v.at[j0 + NBUF + b]],
                                 rows_v.at[b], gsem.at[b])
        return 0
    lax.fori_loop(0, WAVES, wave, 0)

    for b in range(NBUF):
        _drain(ssem.at[b], rows_v.at[b], dummy)

    plsc.subcore_barrier()
    pltpu.sync_copy(acc_sh.at[pl.ds(s * RPT, RPT)],
                    out_hbm.at[c].at[pl.ds(s * RPT, RPT)])


def _tc_mm_body(x2_ref, w1k_ref, h_ref):
    h_ref[...] = jnp.dot(x2_ref[...], w1k_ref[...],
                         preferred_element_type=jnp.float32)


def _tc_scale_body(degp_ref, h_ref, dinv_ref, g_ref):
    deg = degp_ref[0] + degp_ref[1] + 1.0
    dinv = lax.rsqrt(deg)
    dinv_ref[...] = dinv
    hp = jnp.pad(h_ref[...], ((0, PR - NROWS), (0, 0)))
    g_ref[...] = hp * dinv


def _tc_mid_body(acc_ref, dinv_ref, h_ref, b_ref, z_ref, g_ref):
    dinv = dinv_ref[...]
    hp = jnp.pad(h_ref[...], ((0, PR - NROWS), (0, 0)))
    a = (acc_ref[0] + acc_ref[1]) * dinv + (dinv * dinv) * hp + b_ref[...]
    z = jnp.maximum(a, 0.0)
    z_ref[...] = z
    g_ref[...] = z * dinv


def _tc_post_body(acc_ref, dinv_ref, z_ref, w2k_ref, b_ref, out_ref):
    dinv = dinv_ref[...]
    t = (acc_ref[0] + acc_ref[1]) * dinv + (dinv * dinv) * z_ref[...]
    o = jnp.dot(t, w2k_ref[...], preferred_element_type=jnp.float32)
    o = o + b_ref[...]
    pieces = []
    for b in range(16):
        ob = o[:, b * OUT_C:(b + 1) * OUT_C]
        m = jnp.max(ob, axis=1, keepdims=True)
        e = jnp.exp(ob - m)
        pieces.append((ob - m) - jnp.log(jnp.sum(e, axis=1, keepdims=True)))
    out_ref[...] = jnp.concatenate(pieces, axis=1)


_f32 = jnp.float32


def kernel(x, edge_index, W1, b1, W2, b2):
    dstp = edge_index[1].reshape(NC, NS, CH, CHUNK)

    w1k = jnp.kron(jnp.eye(16, dtype=_f32), W1)
    w2k = jnp.kron(jnp.eye(16, dtype=_f32), W2)
    b1k = jnp.tile(b1, 16).reshape(1, PK)
    b2k = jnp.tile(b2, 16).reshape(1, 16 * OUT_C)
    x2 = x.reshape(N // 16, 16 * IN_C)

    degp = _sc_degree(dstp)
    src_row, _ = lax.optimization_barrier((edge_index[0], degp))
    srcp = src_row.reshape(NC, NS, CH, CHUNK)
    h = pl.pallas_call(
        _tc_mm_body,
        out_shape=jax.ShapeDtypeStruct((NROWS, PK), _f32),
    )(x2, w1k)

    dinv, g1p = pl.pallas_call(
        _tc_scale_body,
        out_shape=(jax.ShapeDtypeStruct((PR, PK), _f32),
                   jax.ShapeDtypeStruct((PR, PK), _f32)),
    )(degp.reshape(NC, PR, PK), h)

    acc1 = _sc_aggregate(srcp, dstp, g1p.reshape(NPAD, D))

    z, g2p = pl.pallas_call(
        _tc_mid_body,
        out_shape=(jax.ShapeDtypeStruct((PR, PK), _f32),
                   jax.ShapeDtypeStruct((PR, PK), _f32)),
    )(acc1.reshape(NC, PR, PK), dinv, h, b1k)

    acc2 = _sc_aggregate(srcp, dstp, g2p.reshape(NPAD, D))

    op = pl.pallas_call(
        _tc_post_body,
        out_shape=jax.ShapeDtypeStruct((PR, 16 * OUT_C), _f32),
    )(acc2.reshape(NC, PR, PK), dinv, z, w2k, b2k)
    return op[:NROWS].reshape(N, OUT_C)
